# baseline (speedup 1.0000x reference)
import jax
import jax.numpy as jnp
from jax.experimental import pallas as pl

_BN = 8000


def _drop_path_block(bid_ref, rnd_ref, data_ref, out_ref):
    bid = bid_ref[...]
    rnd_row = rnd_ref[...]
    iota = jax.lax.broadcasted_iota(jnp.int32, (1, 16), 1)
    onehot = bid == iota
    mask = jnp.sum(jnp.where(onehot, rnd_row, 0.0), axis=1, keepdims=True)
    out_ref[...] = data_ref[...] * mask


def kernel(data, batch_id, rnd_tensor, depth):
    n, c = data.shape
    b = rnd_tensor.shape[0]
    bid = batch_id.astype(jnp.int32).reshape(n, 1)
    rnd_row = rnd_tensor.reshape(1, b).astype(jnp.float32)
    grid = (n // _BN,)
    return pl.pallas_call(
        _drop_path_block,
        grid=grid,
        in_specs=[
            pl.BlockSpec((_BN, 1), lambda i: (i, 0)),
            pl.BlockSpec((1, b), lambda i: (0, 0)),
            pl.BlockSpec((_BN, c), lambda i: (i, 0)),
        ],
        out_specs=pl.BlockSpec((_BN, c), lambda i: (i, 0)),
        out_shape=jax.ShapeDtypeStruct((n, c), data.dtype),
    )(bid, rnd_row, data)


# speedup vs baseline: 1.1239x; 1.1239x over previous
import jax
import jax.numpy as jnp
from jax import lax
from jax.experimental import pallas as pl
from jax.experimental.pallas import tpu as pltpu

_G = 1000
_SUBS = 8
_BN = _G * _SUBS
_B = 16


def _drop_path_block(bidc_s, rnd_s, bid_ref, rnd_v_ref, data_ref, out_ref):
    i = pl.program_id(0)
    rows2 = _G // 2

    for s in range(_SUBS):
        k = i * _SUBS + s
        first = bidc_s[k]
        nxt = bidc_s[k + 1]
        r0 = s * rows2

        @pl.when(first == nxt)
        def _uniform():
            m = rnd_s[first]
            out_ref[r0:r0 + rows2, :] = data_ref[r0:r0 + rows2, :] * m

        @pl.when(first != nxt)
        def _straddle():
            slab = bid_ref[0, s].reshape(1, _G)
            biota = lax.broadcasted_iota(jnp.int32, (_B, _G), 0)
            lb = jnp.sum((slab < biota).astype(jnp.int32), axis=1,
                         keepdims=True)
            rv = rnd_v_ref[...]
            d16 = rv - jnp.concatenate(
                [jnp.zeros((1, 1), jnp.float32), rv[:-1]], axis=0)
            j = 2 * lax.broadcasted_iota(jnp.int32, (rows2, 128), 0) + (
                lax.broadcasted_iota(jnp.int32, (rows2, 128), 1) // 64)
            acc = jnp.zeros((rows2, 128), jnp.float32)
            for b in range(_B):
                acc = acc + jnp.where(j >= lb[b:b + 1], d16[b:b + 1], 0.0)
            out_ref[r0:r0 + rows2, :] = data_ref[r0:r0 + rows2, :] * acc


def kernel(data, batch_id, rnd_tensor, depth):
    n, c = data.shape
    bid = batch_id.astype(jnp.int32)
    nblk = n // _BN
    bidc = jnp.concatenate([bid[::_G], bid[-1:]])
    rnd_s = rnd_tensor.reshape(_B).astype(jnp.float32)
    rnd_v = rnd_tensor.reshape(_B, 1).astype(jnp.float32)
    bid3 = bid.reshape(nblk, _SUBS, _G)
    data2 = data.reshape(n * c // 128, 128)
    rows2_blk = _BN * c // 128

    out2 = pl.pallas_call(
        _drop_path_block,
        grid_spec=pltpu.PrefetchScalarGridSpec(
            num_scalar_prefetch=2,
            grid=(nblk,),
            in_specs=[
                pl.BlockSpec((1, _SUBS, _G), lambda i, *_: (i, 0, 0)),
                pl.BlockSpec((_B, 1), lambda i, *_: (0, 0)),
                pl.BlockSpec((rows2_blk, 128), lambda i, *_: (i, 0)),
            ],
            out_specs=pl.BlockSpec((rows2_blk, 128), lambda i, *_: (i, 0)),
        ),
        out_shape=jax.ShapeDtypeStruct((n * c // 128, 128), data.dtype),
        compiler_params=pltpu.CompilerParams(
            dimension_semantics=("parallel",)),
    )(bidc, rnd_s, bid3, rnd_v, data2)
    return out2.reshape(n, c)


# speedup vs baseline: 1.5432x; 1.3731x over previous
import jax
import jax.numpy as jnp
from jax import lax
from jax.experimental import pallas as pl
from jax.experimental.pallas import tpu as pltpu

_G = 500
_SUBS = 16
_BN = _G * _SUBS
_B = 16


def _drop_path_block(bidc_s, rnd_s, bid_ref, rnd_v_ref, data_ref, out_ref):
    i = pl.program_id(0)

    for s in range(_SUBS):
        k = i * _SUBS + s
        first = bidc_s[k]
        nxt = bidc_s[k + 1]
        r0 = s * _G

        @pl.when(first == nxt)
        def _uniform():
            m = rnd_s[first]
            out_ref[r0:r0 + _G, :] = data_ref[r0:r0 + _G, :] * m

        @pl.when(first != nxt)
        def _straddle():
            slab = bid_ref[0, s].reshape(1, _G)
            biota = lax.broadcasted_iota(jnp.int32, (_B, _G), 0)
            lb = jnp.sum((slab < biota).astype(jnp.int32), axis=1,
                         keepdims=True)
            rv = rnd_v_ref[...]
            d16 = rv - jnp.concatenate(
                [jnp.zeros((1, 1), jnp.float32), rv[:-1]], axis=0)
            j = lax.broadcasted_iota(jnp.int32, (_G, data_ref.shape[1]), 0)
            acc = jnp.zeros((_G, data_ref.shape[1]), jnp.float32)
            for b in range(_B):
                acc = acc + jnp.where(j >= lb[b:b + 1], d16[b:b + 1], 0.0)
            out_ref[r0:r0 + _G, :] = data_ref[r0:r0 + _G, :] * acc


def kernel(data, batch_id, rnd_tensor, depth):
    n, c = data.shape
    bid = batch_id.astype(jnp.int32)
    nblk = n // _BN
    bidc = jnp.concatenate([bid[::_G], bid[-1:]])
    rnd_s = rnd_tensor.reshape(_B).astype(jnp.float32)
    rnd_v = rnd_tensor.reshape(_B, 1).astype(jnp.float32)
    bid3 = bid.reshape(nblk, _SUBS, _G)

    return pl.pallas_call(
        _drop_path_block,
        grid_spec=pltpu.PrefetchScalarGridSpec(
            num_scalar_prefetch=2,
            grid=(nblk,),
            in_specs=[
                pl.BlockSpec((1, _SUBS, _G), lambda i, *_: (i, 0, 0)),
                pl.BlockSpec((_B, 1), lambda i, *_: (0, 0)),
                pl.BlockSpec((_BN, c), lambda i, *_: (i, 0)),
            ],
            out_specs=pl.BlockSpec((_BN, c), lambda i, *_: (i, 0)),
        ),
        out_shape=jax.ShapeDtypeStruct((n, c), data.dtype),
        compiler_params=pltpu.CompilerParams(
            dimension_semantics=("parallel",)),
    )(bidc, rnd_s, bid3, rnd_v, data)
